# baseline (speedup 1.0000x reference)
import jax
import jax.numpy as jnp
from jax.experimental import pallas as pl
from jax.experimental.pallas import tpu as pltpu

D_MODEL = 2048
D_SPACE = 64
_POOLS = (1024, 1024, 1024, 1024, 1024, 1024, 2048, 2048)
_EMB_OFF = (0, 1024, 2048, 3072, 4096, 5120, 6144, 8192)
_TOTAL_EMB = 10240
_NPROJ = 8 * D_SPACE
_TM = 256


def _router_body(x_ref, wa_ref, wf_ref, wr_ref, ba_ref, bf_ref, br_ref,
                 emb_ref, *refs):
    out_refs = refs[:8]
    normb_ref, wb_ref, bc_ref = refs[8], refs[9], refs[10]
    b = pl.program_id(0)
    t = pl.program_id(1)

    @pl.when(jnp.logical_and(b == 0, t == 0))
    def _():
        e = emb_ref[...]
        ss = jnp.sum(e * e, axis=1, keepdims=True)
        normb_ref[...] = (e / jnp.maximum(jnp.sqrt(ss), 1e-12)).astype(jnp.bfloat16)
        wb_ref[0:6 * D_SPACE, :] = wa_ref[...].astype(jnp.bfloat16)
        wb_ref[6 * D_SPACE:7 * D_SPACE, :] = wf_ref[...].astype(jnp.bfloat16)
        wb_ref[7 * D_SPACE:8 * D_SPACE, :] = wr_ref[...].astype(jnp.bfloat16)
        bc_ref[:, 0:6 * D_SPACE] = ba_ref[...]
        bc_ref[:, 6 * D_SPACE:7 * D_SPACE] = bf_ref[...]
        bc_ref[:, 7 * D_SPACE:8 * D_SPACE] = br_ref[...]

    h = jax.lax.dot_general(
        x_ref[0].astype(jnp.bfloat16), wb_ref[...], (((1,), (1,)), ((), ())),
        preferred_element_type=jnp.float32) + bc_ref[...]
    hb = h.astype(jnp.bfloat16)
    for p in range(8):
        hp = hb[:, p * D_SPACE:(p + 1) * D_SPACE]
        ep = normb_ref[_EMB_OFF[p]:_EMB_OFF[p] + _POOLS[p], :]
        out_refs[p][0] = jax.lax.dot_general(
            hp, ep, (((1,), (1,)), ((), ())),
            preferred_element_type=jnp.float32)


def kernel(x, W_all, b_all, W_fk, b_fk, W_rk, b_rk, neuron_emb):
    B, S, D = x.shape

    grid = (B, S // _TM)
    outs = pl.pallas_call(
        _router_body,
        grid=grid,
        in_specs=[
            pl.BlockSpec((1, _TM, D_MODEL), lambda b, t: (b, t, 0)),
            pl.BlockSpec((6 * D_SPACE, D_MODEL), lambda b, t: (0, 0)),
            pl.BlockSpec((D_SPACE, D_MODEL), lambda b, t: (0, 0)),
            pl.BlockSpec((D_SPACE, D_MODEL), lambda b, t: (0, 0)),
            pl.BlockSpec((1, 6 * D_SPACE), lambda b, t: (0, 0)),
            pl.BlockSpec((1, D_SPACE), lambda b, t: (0, 0)),
            pl.BlockSpec((1, D_SPACE), lambda b, t: (0, 0)),
            pl.BlockSpec((_TOTAL_EMB, D_SPACE), lambda b, t: (0, 0)),
        ],
        out_specs=[pl.BlockSpec((1, _TM, n), lambda b, t: (b, t, 0))
                   for n in _POOLS],
        out_shape=[jax.ShapeDtypeStruct((B, S, n), jnp.float32) for n in _POOLS],
        scratch_shapes=[pltpu.VMEM((_TOTAL_EMB, D_SPACE), jnp.bfloat16),
                        pltpu.VMEM((_NPROJ, D_MODEL), jnp.bfloat16),
                        pltpu.VMEM((1, _NPROJ), jnp.float32)],
    )(x, W_all, W_fk, W_rk, b_all[None, :], b_fk[None, :], b_rk[None, :],
      neuron_emb)
    return outs


# speedup vs baseline: 1.1138x; 1.1138x over previous
import jax
import jax.numpy as jnp
from jax.experimental import pallas as pl
from jax.experimental.pallas import tpu as pltpu

D_MODEL = 2048
D_SPACE = 64
_POOLS = (1024, 1024, 1024, 1024, 1024, 1024, 2048, 2048)
_EMB_OFF = (0, 1024, 2048, 3072, 4096, 5120, 6144, 8192)
_TOTAL_EMB = 10240
_NPROJ = 8 * D_SPACE
_TM = 256


def _router_body(x_ref, wa_ref, wf_ref, wr_ref, ba_ref, bf_ref, br_ref,
                 embt_ref, *refs):
    out_refs = refs[:8]
    normt_ref, wb_ref, bc_ref = refs[8], refs[9], refs[10]
    b = pl.program_id(0)
    t = pl.program_id(1)

    @pl.when(jnp.logical_and(b == 0, t == 0))
    def _():
        e = embt_ref[...]
        ss = jnp.sum(e * e, axis=0, keepdims=True)
        normt_ref[...] = (e / jnp.maximum(jnp.sqrt(ss), 1e-12)).astype(jnp.bfloat16)
        wb_ref[0:6 * D_SPACE, :] = wa_ref[...].astype(jnp.bfloat16)
        wb_ref[6 * D_SPACE:7 * D_SPACE, :] = wf_ref[...].astype(jnp.bfloat16)
        wb_ref[7 * D_SPACE:8 * D_SPACE, :] = wr_ref[...].astype(jnp.bfloat16)
        bc_ref[:, 0:6 * D_SPACE] = ba_ref[...]
        bc_ref[:, 6 * D_SPACE:7 * D_SPACE] = bf_ref[...]
        bc_ref[:, 7 * D_SPACE:8 * D_SPACE] = br_ref[...]

    h = jax.lax.dot_general(
        x_ref[0].astype(jnp.bfloat16), wb_ref[...], (((1,), (1,)), ((), ())),
        preferred_element_type=jnp.float32) + bc_ref[...]
    hb = h.astype(jnp.bfloat16)
    for p in range(8):
        hp = hb[:, p * D_SPACE:(p + 1) * D_SPACE]
        ep = normt_ref[:, _EMB_OFF[p]:_EMB_OFF[p] + _POOLS[p]]
        out_refs[p][0] = jax.lax.dot_general(
            hp, ep, (((1,), (0,)), ((), ())),
            preferred_element_type=jnp.float32)


def kernel(x, W_all, b_all, W_fk, b_fk, W_rk, b_rk, neuron_emb):
    B, S, D = x.shape

    grid = (B, S // _TM)
    outs = pl.pallas_call(
        _router_body,
        grid=grid,
        in_specs=[
            pl.BlockSpec((1, _TM, D_MODEL), lambda b, t: (b, t, 0)),
            pl.BlockSpec((6 * D_SPACE, D_MODEL), lambda b, t: (0, 0)),
            pl.BlockSpec((D_SPACE, D_MODEL), lambda b, t: (0, 0)),
            pl.BlockSpec((D_SPACE, D_MODEL), lambda b, t: (0, 0)),
            pl.BlockSpec((1, 6 * D_SPACE), lambda b, t: (0, 0)),
            pl.BlockSpec((1, D_SPACE), lambda b, t: (0, 0)),
            pl.BlockSpec((1, D_SPACE), lambda b, t: (0, 0)),
            pl.BlockSpec((D_SPACE, _TOTAL_EMB), lambda b, t: (0, 0)),
        ],
        out_specs=[pl.BlockSpec((1, _TM, n), lambda b, t: (b, t, 0))
                   for n in _POOLS],
        out_shape=[jax.ShapeDtypeStruct((B, S, n), jnp.float32) for n in _POOLS],
        scratch_shapes=[pltpu.VMEM((D_SPACE, _TOTAL_EMB), jnp.bfloat16),
                        pltpu.VMEM((_NPROJ, D_MODEL), jnp.bfloat16),
                        pltpu.VMEM((1, _NPROJ), jnp.float32)],
    )(x, W_all, W_fk, W_rk, b_all[None, :], b_fk[None, :], b_rk[None, :],
      neuron_emb.T)
    return outs


# speedup vs baseline: 1.1332x; 1.0174x over previous
import jax
import jax.numpy as jnp
from jax.experimental import pallas as pl
from jax.experimental.pallas import tpu as pltpu

D_MODEL = 2048
D_SPACE = 64
_POOLS = (1024, 1024, 1024, 1024, 1024, 1024, 2048, 2048)
_EMB_OFF = (0, 1024, 2048, 3072, 4096, 5120, 6144, 8192)
_TOTAL_EMB = 10240
_NPROJ = 8 * D_SPACE
_TM = 256


def _router_body(x_ref, wa_ref, wf_ref, wr_ref, ba_ref, bf_ref, br_ref,
                 embt_ref, *refs):
    out_refs = refs[:8]
    normt_ref, wb_ref, bc_ref = refs[8], refs[9], refs[10]
    b = pl.program_id(0)
    t = pl.program_id(1)

    @pl.when(jnp.logical_and(b == 0, t == 0))
    def _():
        e = embt_ref[...]
        ss = jnp.sum(e * e, axis=0, keepdims=True)
        normt_ref[...] = (e / jnp.maximum(jnp.sqrt(ss), 1e-12)).astype(jnp.bfloat16)
        wb_ref[0:6 * D_SPACE, :] = wa_ref[...].astype(jnp.bfloat16)
        wb_ref[6 * D_SPACE:7 * D_SPACE, :] = wf_ref[...].astype(jnp.bfloat16)
        wb_ref[7 * D_SPACE:8 * D_SPACE, :] = wr_ref[...].astype(jnp.bfloat16)
        bc_ref[:, 0:6 * D_SPACE] = ba_ref[...].reshape(1, 6 * D_SPACE)
        bc_ref[:, 6 * D_SPACE:7 * D_SPACE] = bf_ref[...].reshape(1, D_SPACE)
        bc_ref[:, 7 * D_SPACE:8 * D_SPACE] = br_ref[...].reshape(1, D_SPACE)

    h = jax.lax.dot_general(
        x_ref[0].astype(jnp.bfloat16), wb_ref[...], (((1,), (1,)), ((), ())),
        preferred_element_type=jnp.float32) + bc_ref[...]
    hb = h.astype(jnp.bfloat16)
    for p in range(8):
        hp = hb[:, p * D_SPACE:(p + 1) * D_SPACE]
        ep = normt_ref[:, _EMB_OFF[p]:_EMB_OFF[p] + _POOLS[p]]
        out_refs[p][0] = jax.lax.dot_general(
            hp, ep, (((1,), (0,)), ((), ())),
            preferred_element_type=jnp.float32)


def kernel(x, W_all, b_all, W_fk, b_fk, W_rk, b_rk, neuron_emb):
    B, S, D = x.shape

    grid = (B, S // _TM)
    outs = pl.pallas_call(
        _router_body,
        grid=grid,
        in_specs=[
            pl.BlockSpec((1, _TM, D_MODEL), lambda b, t: (b, t, 0)),
            pl.BlockSpec((6 * D_SPACE, D_MODEL), lambda b, t: (0, 0)),
            pl.BlockSpec((D_SPACE, D_MODEL), lambda b, t: (0, 0)),
            pl.BlockSpec((D_SPACE, D_MODEL), lambda b, t: (0, 0)),
            pl.BlockSpec((6 * D_SPACE,), lambda b, t: (0,)),
            pl.BlockSpec((D_SPACE,), lambda b, t: (0,)),
            pl.BlockSpec((D_SPACE,), lambda b, t: (0,)),
            pl.BlockSpec((D_SPACE, _TOTAL_EMB), lambda b, t: (0, 0)),
        ],
        out_specs=[pl.BlockSpec((1, _TM, n), lambda b, t: (b, t, 0))
                   for n in _POOLS],
        out_shape=[jax.ShapeDtypeStruct((B, S, n), jnp.float32) for n in _POOLS],
        scratch_shapes=[pltpu.VMEM((D_SPACE, _TOTAL_EMB), jnp.bfloat16),
                        pltpu.VMEM((_NPROJ, D_MODEL), jnp.bfloat16),
                        pltpu.VMEM((1, _NPROJ), jnp.float32)],
    )(x, W_all, W_fk, W_rk, b_all, b_fk, b_rk, neuron_emb.T)
    return outs


# speedup vs baseline: 1.1362x; 1.0026x over previous
import jax
import jax.numpy as jnp
from jax.experimental import pallas as pl
from jax.experimental.pallas import tpu as pltpu

D_MODEL = 2048
D_SPACE = 64
_POOLS = (1024, 1024, 1024, 1024, 1024, 1024, 2048, 2048)
_EMB_OFF = (0, 1024, 2048, 3072, 4096, 5120, 6144, 8192)
_TOTAL_EMB = 10240
_NPROJ = 8 * D_SPACE
_TM = 256


def _router_body(x_ref, wa_ref, wf_ref, wr_ref, ba_ref, bf_ref, br_ref,
                 embt_ref, *refs):
    out_refs = refs[:8]
    normt_ref, wb_ref, bc_ref = refs[8], refs[9], refs[10]
    b = pl.program_id(0)
    t = pl.program_id(1)

    del b
    @pl.when(t == 0)
    def _():
        e = embt_ref[...]
        ss = jnp.sum(e * e, axis=0, keepdims=True)
        normt_ref[...] = (e / jnp.maximum(jnp.sqrt(ss), 1e-12)).astype(jnp.bfloat16)
        wb_ref[0:6 * D_SPACE, :] = wa_ref[...].astype(jnp.bfloat16)
        wb_ref[6 * D_SPACE:7 * D_SPACE, :] = wf_ref[...].astype(jnp.bfloat16)
        wb_ref[7 * D_SPACE:8 * D_SPACE, :] = wr_ref[...].astype(jnp.bfloat16)
        bc_ref[:, 0:6 * D_SPACE] = ba_ref[...].reshape(1, 6 * D_SPACE)
        bc_ref[:, 6 * D_SPACE:7 * D_SPACE] = bf_ref[...].reshape(1, D_SPACE)
        bc_ref[:, 7 * D_SPACE:8 * D_SPACE] = br_ref[...].reshape(1, D_SPACE)

    h = jax.lax.dot_general(
        x_ref[0].astype(jnp.bfloat16), wb_ref[...], (((1,), (1,)), ((), ())),
        preferred_element_type=jnp.float32) + bc_ref[...]
    hb = h.astype(jnp.bfloat16)
    for p in range(8):
        hp = hb[:, p * D_SPACE:(p + 1) * D_SPACE]
        ep = normt_ref[:, _EMB_OFF[p]:_EMB_OFF[p] + _POOLS[p]]
        out_refs[p][0] = jax.lax.dot_general(
            hp, ep, (((1,), (0,)), ((), ())),
            preferred_element_type=jnp.float32)


def kernel(x, W_all, b_all, W_fk, b_fk, W_rk, b_rk, neuron_emb):
    B, S, D = x.shape

    grid = (B, S // _TM)
    outs = pl.pallas_call(
        _router_body,
        grid=grid,
        in_specs=[
            pl.BlockSpec((1, _TM, D_MODEL), lambda b, t: (b, t, 0)),
            pl.BlockSpec((6 * D_SPACE, D_MODEL), lambda b, t: (0, 0)),
            pl.BlockSpec((D_SPACE, D_MODEL), lambda b, t: (0, 0)),
            pl.BlockSpec((D_SPACE, D_MODEL), lambda b, t: (0, 0)),
            pl.BlockSpec((6 * D_SPACE,), lambda b, t: (0,)),
            pl.BlockSpec((D_SPACE,), lambda b, t: (0,)),
            pl.BlockSpec((D_SPACE,), lambda b, t: (0,)),
            pl.BlockSpec((D_SPACE, _TOTAL_EMB), lambda b, t: (0, 0)),
        ],
        out_specs=[pl.BlockSpec((1, _TM, n), lambda b, t: (b, t, 0))
                   for n in _POOLS],
        out_shape=[jax.ShapeDtypeStruct((B, S, n), jnp.float32) for n in _POOLS],
        scratch_shapes=[pltpu.VMEM((D_SPACE, _TOTAL_EMB), jnp.bfloat16),
                        pltpu.VMEM((_NPROJ, D_MODEL), jnp.bfloat16),
                        pltpu.VMEM((1, _NPROJ), jnp.float32)],
        compiler_params=pltpu.CompilerParams(
            dimension_semantics=("parallel", "arbitrary")),
    )(x, W_all, W_fk, W_rk, b_all, b_fk, b_rk, neuron_emb.T)
    return outs


# speedup vs baseline: 1.1459x; 1.0086x over previous
import jax
import jax.numpy as jnp
from jax.experimental import pallas as pl
from jax.experimental.pallas import tpu as pltpu

D_MODEL = 2048
D_SPACE = 64
_POOLS = (1024, 1024, 1024, 1024, 1024, 1024, 2048, 2048)
_EMB_OFF = (0, 1024, 2048, 3072, 4096, 5120, 6144, 8192)
_TOTAL_EMB = 10240
_NPROJ = 8 * D_SPACE
_TM = 512


def _router_body(x_ref, wa_ref, wf_ref, wr_ref, ba_ref, bf_ref, br_ref,
                 embt_ref, *refs):
    out_refs = refs[:8]
    normt_ref, wb_ref, bc_ref = refs[8], refs[9], refs[10]
    b = pl.program_id(0)
    t = pl.program_id(1)

    @pl.when(jnp.logical_and(b == 0, t == 0))
    def _():
        e = embt_ref[...]
        ss = jnp.sum(e * e, axis=0, keepdims=True)
        normt_ref[...] = (e / jnp.maximum(jnp.sqrt(ss), 1e-12)).astype(jnp.bfloat16)
        wb_ref[0:6 * D_SPACE, :] = wa_ref[...].astype(jnp.bfloat16)
        wb_ref[6 * D_SPACE:7 * D_SPACE, :] = wf_ref[...].astype(jnp.bfloat16)
        wb_ref[7 * D_SPACE:8 * D_SPACE, :] = wr_ref[...].astype(jnp.bfloat16)
        bc_ref[:, 0:6 * D_SPACE] = ba_ref[...].reshape(1, 6 * D_SPACE)
        bc_ref[:, 6 * D_SPACE:7 * D_SPACE] = bf_ref[...].reshape(1, D_SPACE)
        bc_ref[:, 7 * D_SPACE:8 * D_SPACE] = br_ref[...].reshape(1, D_SPACE)

    h = jax.lax.dot_general(
        x_ref[0].astype(jnp.bfloat16), wb_ref[...], (((1,), (1,)), ((), ())),
        preferred_element_type=jnp.float32) + bc_ref[...]
    hb = h.astype(jnp.bfloat16)
    for p in range(8):
        hp = hb[:, p * D_SPACE:(p + 1) * D_SPACE]
        ep = normt_ref[:, _EMB_OFF[p]:_EMB_OFF[p] + _POOLS[p]]
        out_refs[p][0] = jax.lax.dot_general(
            hp, ep, (((1,), (0,)), ((), ())),
            preferred_element_type=jnp.float32)


def kernel(x, W_all, b_all, W_fk, b_fk, W_rk, b_rk, neuron_emb):
    B, S, D = x.shape

    grid = (B, S // _TM)
    outs = pl.pallas_call(
        _router_body,
        grid=grid,
        in_specs=[
            pl.BlockSpec((1, _TM, D_MODEL), lambda b, t: (b, t, 0)),
            pl.BlockSpec((6 * D_SPACE, D_MODEL), lambda b, t: (0, 0)),
            pl.BlockSpec((D_SPACE, D_MODEL), lambda b, t: (0, 0)),
            pl.BlockSpec((D_SPACE, D_MODEL), lambda b, t: (0, 0)),
            pl.BlockSpec((6 * D_SPACE,), lambda b, t: (0,)),
            pl.BlockSpec((D_SPACE,), lambda b, t: (0,)),
            pl.BlockSpec((D_SPACE,), lambda b, t: (0,)),
            pl.BlockSpec((D_SPACE, _TOTAL_EMB), lambda b, t: (0, 0)),
        ],
        out_specs=[pl.BlockSpec((1, _TM, n), lambda b, t: (b, t, 0))
                   for n in _POOLS],
        out_shape=[jax.ShapeDtypeStruct((B, S, n), jnp.float32) for n in _POOLS],
        scratch_shapes=[pltpu.VMEM((D_SPACE, _TOTAL_EMB), jnp.bfloat16),
                        pltpu.VMEM((_NPROJ, D_MODEL), jnp.bfloat16),
                        pltpu.VMEM((1, _NPROJ), jnp.float32)],
        compiler_params=pltpu.CompilerParams(
            vmem_limit_bytes=100 * 1024 * 1024),
    )(x, W_all, W_fk, W_rk, b_all, b_fk, b_rk, neuron_emb.T)
    return outs
